# Initial kernel scaffold; baseline (speedup 1.0000x reference)
import jax
import jax.numpy as jnp
from jax.experimental import pallas as pl


def kernel(xyzt, hash_table):
    raise NotImplementedError("write your pallas kernel here")


# speedup vs baseline: 7.3756x; 7.3756x over previous
import functools

import numpy as np
import jax
import jax.numpy as jnp
from jax import lax
from jax.experimental import pallas as pl
from jax.experimental.pallas import tpu as pltpu, tpu_sc as plsc

_MIN_RES = np.array([16, 16, 16, 16], dtype=np.float64)
_MAX_RES = np.array([256, 256, 256, 128], dtype=np.float64)
_NUM_SCALES = 16
_MAX_PARAMS = 2 ** 19
_F = 2
_PRIMES = (1, 2654435761, 805459861, 3674653429)


def _levels():
    b = np.exp((np.log(_MAX_RES) - np.log(_MIN_RES)) / (_NUM_SCALES - 1))
    out, total = [], 0
    for i in range(_NUM_SCALES):
        res = np.ceil(_MIN_RES * np.power(b, i)).astype(np.int64)
        raw = int(res[0] + 1) * int(res[1] + 1) * int(res[2] + 1) * int(res[3] + 1)
        p = raw if raw % 8 == 0 else (raw + 7) // 8 * 8
        p = min(_MAX_PARAMS, p)
        out.append(dict(res=tuple(int(r) for r in res), size=p, tiled=raw <= p,
                        off=total))
        total += p * _F
    return out, total


_LV, _TOTAL = _levels()

_NC = 2
_NS = 16
_NW = _NC * _NS
_P = 512
_G = _P // 16
_CP = 16 * _P
_N = 1000000
_NCHUNK = -(-_N // (_NW * _P))
_PW = _NCHUNK * _P
_NPAD = _NW * _PW


def _tec_kernel(xyzt_hbm, table_hbm, out_hbm, xyzt_v, frac_v, idx_v, rows_v,
                out_v, sem):
    wid = lax.axis_index("s") * _NC + lax.axis_index("c")
    base_pt = wid * _PW
    iota = lax.iota(jnp.int32, 16)

    def chunk_body(ci, carry):
        pt0 = base_pt + ci * _P
        pltpu.sync_copy(xyzt_hbm.at[:, pl.ds(pt0, _P)], xyzt_v)
        for s, lv in enumerate(_LV):
            resf = [np.float32(r) for r in lv["res"]]

            def build_a(g, c2, lv=lv, resf=resf):
                b16 = g * 16
                pg, fr = [], []
                for d in range(4):
                    pos = xyzt_v[d, pl.ds(b16, 16)] * resf[d]
                    pgi = pos.astype(jnp.int32)
                    fr_d = pos - pgi.astype(jnp.float32)
                    pg.append(pgi)
                    fr.append(fr_d)
                    frac_v[d, pl.ds(b16, 16)] = fr_d
                if lv["tiled"]:
                    r = lv["res"]
                    st = (1, r[0] + 1, (r[0] + 1) * (r[1] + 1),
                          (r[0] + 1) * (r[1] + 1) * (r[2] + 1))
                    lo = [pg[d] * jnp.int32(2 * st[d]) for d in range(4)]
                    lo[0] = lo[0] + jnp.int32(lv["off"])
                    hi = [lo[d] + jnp.int32(2 * st[d]) for d in range(4)]
                    for c in range(16):
                        t = [hi[d] if (c >> d) & 1 else lo[d] for d in range(4)]
                        e0 = (t[0] + t[1]) + (t[2] + t[3])
                        idx_v[pl.ds(c * _P + b16, 16)] = e0
                        idx_v[pl.ds(_CP + c * _P + b16, 16)] = e0 + 1
                else:
                    mask = jnp.uint32(lv["size"] - 1)
                    off = jnp.int32(lv["off"])
                    lo = [pg[d].astype(jnp.uint32) * jnp.uint32(_PRIMES[d])
                          for d in range(4)]
                    hi = [lo[d] + jnp.uint32(_PRIMES[d]) for d in range(4)]
                    for c in range(16):
                        t = [hi[d] if (c >> d) & 1 else lo[d] for d in range(4)]
                        h = ((t[0] ^ t[1]) ^ (t[2] ^ t[3])) & mask
                        e0 = h.astype(jnp.int32) * 2 + off
                        idx_v[pl.ds(c * _P + b16, 16)] = e0
                        idx_v[pl.ds(_CP + c * _P + b16, 16)] = e0 + 1
                return c2

            lax.fori_loop(0, _G, build_a, 0, unroll=False)
            pltpu.async_copy(table_hbm.at[idx_v], rows_v, sem).wait()

            def pass_b(g, c2, s=s):
                b16 = g * 16
                fr = [frac_v[d, pl.ds(b16, 16)] for d in range(4)]
                om = [1.0 - f for f in fr]
                u = [om[0] * om[1], fr[0] * om[1], om[0] * fr[1], fr[0] * fr[1]]
                v = [om[2] * om[3], fr[2] * om[3], om[2] * fr[3], fr[2] * fr[3]]
                acc0 = jnp.zeros((16,), jnp.float32)
                acc1 = jnp.zeros((16,), jnp.float32)
                for c in range(16):
                    w = u[c & 3] * v[c >> 2]
                    g0 = rows_v[pl.ds(c * _P + b16, 16)]
                    g1 = rows_v[pl.ds(_CP + c * _P + b16, 16)]
                    acc0 = acc0 + w * g0
                    acc1 = acc1 + w * g1
                oidx = (b16 + iota) * 32 + (2 * s)
                plsc.store_scatter(out_v, [oidx], acc0)
                plsc.store_scatter(out_v, [oidx + 1], acc1)
                return c2

            lax.fori_loop(0, _G, pass_b, 0, unroll=False)
        pltpu.sync_copy(out_v, out_hbm.at[pl.ds(pt0 * 32, _P * 32)])
        return carry

    lax.fori_loop(0, _NCHUNK, chunk_body, 0, unroll=False)


@jax.jit
def _encode(xyzt_t, table):
    mesh = plsc.VectorSubcoreMesh(core_axis_name="c", subcore_axis_name="s",
                                  num_cores=_NC, num_subcores=_NS)
    f = pl.kernel(
        _tec_kernel,
        out_type=jax.ShapeDtypeStruct((_NPAD * 2 * _NUM_SCALES,), jnp.float32),
        mesh=mesh,
        compiler_params=pltpu.CompilerParams(needs_layout_passes=False),
        scratch_types=[
            pltpu.VMEM((4, _P), jnp.float32),
            pltpu.VMEM((4, _P), jnp.float32),
            pltpu.VMEM((2 * _CP,), jnp.int32),
            pltpu.VMEM((2 * _CP,), jnp.float32),
            pltpu.VMEM((_P * 2 * _NUM_SCALES,), jnp.float32),
            pltpu.SemaphoreType.DMA,
        ],
    )
    return f(xyzt_t, table)


def kernel(xyzt, hash_table):
    n = xyzt.shape[0]
    xyzt_t = jnp.zeros((4, _NPAD), jnp.float32).at[:, :n].set(xyzt.T)
    out = _encode(xyzt_t, hash_table)
    return out.reshape(_NPAD, 2 * _NUM_SCALES)[:n]


# speedup vs baseline: 10.3634x; 1.4051x over previous
import functools

import numpy as np
import jax
import jax.numpy as jnp
from jax import lax
from jax.experimental import pallas as pl
from jax.experimental.pallas import tpu as pltpu, tpu_sc as plsc

_MIN_RES = np.array([16, 16, 16, 16], dtype=np.float64)
_MAX_RES = np.array([256, 256, 256, 128], dtype=np.float64)
_NUM_SCALES = 16
_MAX_PARAMS = 2 ** 19
_F = 2
_PRIMES = (1, 2654435761, 805459861, 3674653429)


def _levels():
    b = np.exp((np.log(_MAX_RES) - np.log(_MIN_RES)) / (_NUM_SCALES - 1))
    out, total = [], 0
    for i in range(_NUM_SCALES):
        res = np.ceil(_MIN_RES * np.power(b, i)).astype(np.int64)
        raw = int(res[0] + 1) * int(res[1] + 1) * int(res[2] + 1) * int(res[3] + 1)
        p = raw if raw % 8 == 0 else (raw + 7) // 8 * 8
        p = min(_MAX_PARAMS, p)
        out.append(dict(res=tuple(int(r) for r in res), size=p, tiled=raw <= p,
                        row_off=total // 2))
        total += p * _F
    return out, total


_LV, _TOTAL = _levels()

_NC = 2
_NS = 16
_NW = _NC * _NS
_P = 512
_G = _P // 16
_CP = 16 * _P
_N = 1000000
_NCHUNK = -(-_N // (_NW * _P))
_PW = _NCHUNK * _P
_NPAD = _NW * _PW


def _tec_kernel(xyzt_hbm, table_hbm, out_hbm, xyzt_v, frac_v, idx0_v, idx1_v,
                rows0_v, rows1_v, out_v, sem0, sem1):
    wid = lax.axis_index("s") * _NC + lax.axis_index("c")
    base_pt = wid * _PW
    iota = lax.iota(jnp.int32, 16)
    col0 = jnp.zeros((16,), jnp.int32)
    col1 = jnp.ones((16,), jnp.int32)
    sems = (sem0, sem1)
    idxs = (idx0_v, idx1_v)
    rows = (rows0_v, rows1_v)

    def build_a(g, c2, lv, fb):
        b16 = g * 16
        pg = []
        for d in range(4):
            pos = xyzt_v[d, pl.ds(b16, 16)] * np.float32(lv["res"][d])
            pgi = pos.astype(jnp.int32)
            frac_v[fb, d, pl.ds(b16, 16)] = pos - pgi.astype(jnp.float32)
            pg.append(pgi)
        if lv["tiled"]:
            r = lv["res"]
            st = (1, r[0] + 1, (r[0] + 1) * (r[1] + 1),
                  (r[0] + 1) * (r[1] + 1) * (r[2] + 1))
            lo = [pg[d] * jnp.int32(2 * st[d]) for d in range(4)]
            lo[0] = lo[0] + jnp.int32(2 * lv["row_off"])
            hi = [lo[d] + jnp.int32(2 * st[d]) for d in range(4)]
            for c in range(16):
                t = [hi[d] if (c >> d) & 1 else lo[d] for d in range(4)]
                e0 = (t[0] + t[1]) + (t[2] + t[3])
                idxs[fb][pl.ds(c * _P + b16, 16)] = e0
                idxs[fb][pl.ds(_CP + c * _P + b16, 16)] = e0 + 1
        else:
            mask = jnp.uint32(lv["size"] - 1)
            off = jnp.int32(2 * lv["row_off"])
            lo = [pg[d].astype(jnp.uint32) * jnp.uint32(_PRIMES[d])
                  for d in range(4)]
            hi = [lo[d] + jnp.uint32(_PRIMES[d]) for d in range(4)]
            for c in range(16):
                t = [hi[d] if (c >> d) & 1 else lo[d] for d in range(4)]
                h = ((t[0] ^ t[1]) ^ (t[2] ^ t[3])) & mask
                e0 = h.astype(jnp.int32) * 2 + off
                idxs[fb][pl.ds(c * _P + b16, 16)] = e0
                idxs[fb][pl.ds(_CP + c * _P + b16, 16)] = e0 + 1
        return c2

    def pass_b(g, c2, s, fb):
        b16 = g * 16
        fr = [frac_v[fb, d, pl.ds(b16, 16)] for d in range(4)]
        om = [1.0 - f for f in fr]
        u = [om[0] * om[1], fr[0] * om[1], om[0] * fr[1], fr[0] * fr[1]]
        v = [om[2] * om[3], fr[2] * om[3], om[2] * fr[3], fr[2] * fr[3]]
        acc0 = jnp.zeros((16,), jnp.float32)
        acc1 = jnp.zeros((16,), jnp.float32)
        rv = rows[fb]
        for c in range(16):
            w = u[c & 3] * v[c >> 2]
            g0 = rv[pl.ds(c * _P + b16, 16)]
            g1 = rv[pl.ds(_CP + c * _P + b16, 16)]
            acc0 = acc0 + w * g0
            acc1 = acc1 + w * g1
        oidx = (b16 + iota) * 32 + (2 * s)
        plsc.store_scatter(out_v, [oidx], acc0)
        plsc.store_scatter(out_v, [oidx + 1], acc1)
        return c2

    def chunk_body(ci, carry):
        pt0 = base_pt + ci * _P
        pltpu.sync_copy(xyzt_hbm.at[:, pl.ds(pt0, _P)], xyzt_v)
        lax.fori_loop(0, _G, functools.partial(build_a, lv=_LV[0], fb=0), 0,
                      unroll=False)
        cp = pltpu.async_copy(table_hbm.at[idx0_v], rows0_v, sem0)
        for s in range(1, _NUM_SCALES + 1):
            fb, pb = s % 2, (s - 1) % 2
            if s < _NUM_SCALES:
                lax.fori_loop(0, _G,
                              functools.partial(build_a, lv=_LV[s], fb=fb), 0,
                              unroll=False)
                nxt = pltpu.async_copy(table_hbm.at[idxs[fb]],
                                       rows[fb], sems[fb])
            cp.wait()
            lax.fori_loop(0, _G, functools.partial(pass_b, s=s - 1, fb=pb), 0,
                          unroll=False)
            if s < _NUM_SCALES:
                cp = nxt
        pltpu.sync_copy(out_v, out_hbm.at[pl.ds(pt0 * 32, _P * 32)])
        return carry

    lax.fori_loop(0, _NCHUNK, chunk_body, 0, unroll=False)


@jax.jit
def _encode(xyzt_t, table2):
    mesh = plsc.VectorSubcoreMesh(core_axis_name="c", subcore_axis_name="s",
                                  num_cores=_NC, num_subcores=_NS)
    f = pl.kernel(
        _tec_kernel,
        out_type=jax.ShapeDtypeStruct((_NPAD * 2 * _NUM_SCALES,), jnp.float32),
        mesh=mesh,
        compiler_params=pltpu.CompilerParams(needs_layout_passes=False),
        scratch_types=[
            pltpu.VMEM((4, _P), jnp.float32),
            pltpu.VMEM((2, 4, _P), jnp.float32),
            pltpu.VMEM((2 * _CP,), jnp.int32),
            pltpu.VMEM((2 * _CP,), jnp.int32),
            pltpu.VMEM((2 * _CP,), jnp.float32),
            pltpu.VMEM((2 * _CP,), jnp.float32),
            pltpu.VMEM((_P * 2 * _NUM_SCALES,), jnp.float32),
            pltpu.SemaphoreType.DMA,
            pltpu.SemaphoreType.DMA,
        ],
    )
    return f(xyzt_t, table2)


def kernel(xyzt, hash_table):
    n = xyzt.shape[0]
    xyzt_t = jnp.zeros((4, _NPAD), jnp.float32).at[:, :n].set(xyzt.T)
    out = _encode(xyzt_t, hash_table)
    return out.reshape(_NPAD, 2 * _NUM_SCALES)[:n]


# speedup vs baseline: 11.1567x; 1.0765x over previous
import functools

import numpy as np
import jax
import jax.numpy as jnp
from jax import lax
from jax.experimental import pallas as pl
from jax.experimental.pallas import tpu as pltpu, tpu_sc as plsc

_MIN_RES = np.array([16, 16, 16, 16], dtype=np.float64)
_MAX_RES = np.array([256, 256, 256, 128], dtype=np.float64)
_NUM_SCALES = 16
_MAX_PARAMS = 2 ** 19
_F = 2
_PRIMES = (1, 2654435761, 805459861, 3674653429)


def _levels():
    b = np.exp((np.log(_MAX_RES) - np.log(_MIN_RES)) / (_NUM_SCALES - 1))
    out, total = [], 0
    for i in range(_NUM_SCALES):
        res = np.ceil(_MIN_RES * np.power(b, i)).astype(np.int64)
        raw = int(res[0] + 1) * int(res[1] + 1) * int(res[2] + 1) * int(res[3] + 1)
        p = raw if raw % 8 == 0 else (raw + 7) // 8 * 8
        p = min(_MAX_PARAMS, p)
        out.append(dict(res=tuple(int(r) for r in res), size=p, tiled=raw <= p,
                        row_off=total // 2))
        total += p * _F
    return out, total


_LV, _TOTAL = _levels()

_NC = 2
_NS = 16
_NW = _NC * _NS
_P = 512
_G = _P // 16
_CP = 16 * _P
_N = 1000000
_NCHUNK = -(-_N // (_NW * _P))
_NPAD = _NW * _NCHUNK * _P
_NCHUNK0 = 80
_NCHUNK1 = 2 * _NCHUNK - _NCHUNK0


def _tec_kernel(xyzt_hbm, table_hbm, out_hbm, xyzt_v, frac_v, idx0_v, idx1_v,
                rows0_v, rows1_v, out_v, sem0, sem1):
    cid = lax.axis_index("c")
    sid = lax.axis_index("s")
    base_chunk = jnp.where(cid == 0, sid * _NCHUNK0,
                           16 * _NCHUNK0 + sid * _NCHUNK1)
    my_chunks = jnp.where(cid == 0, _NCHUNK0, _NCHUNK1)
    iota = lax.iota(jnp.int32, 16)
    col0 = jnp.zeros((16,), jnp.int32)
    col1 = jnp.ones((16,), jnp.int32)
    sems = (sem0, sem1)
    idxs = (idx0_v, idx1_v)
    rows = (rows0_v, rows1_v)

    def build_a(g, c2, lv, fb):
        b16 = g * 16
        pg = []
        for d in range(4):
            pos = xyzt_v[d, pl.ds(b16, 16)] * np.float32(lv["res"][d])
            pgi = pos.astype(jnp.int32)
            frac_v[fb, d, pl.ds(b16, 16)] = pos - pgi.astype(jnp.float32)
            pg.append(pgi)
        if lv["tiled"]:
            r = lv["res"]
            st = (1, r[0] + 1, (r[0] + 1) * (r[1] + 1),
                  (r[0] + 1) * (r[1] + 1) * (r[2] + 1))
            lo = [pg[d] * jnp.int32(2 * st[d]) for d in range(4)]
            lo[0] = lo[0] + jnp.int32(2 * lv["row_off"])
            hi = [lo[d] + jnp.int32(2 * st[d]) for d in range(4)]
            for c in range(16):
                t = [hi[d] if (c >> d) & 1 else lo[d] for d in range(4)]
                e0 = (t[0] + t[1]) + (t[2] + t[3])
                idxs[fb][pl.ds(c * _P + b16, 16)] = e0
                idxs[fb][pl.ds(_CP + c * _P + b16, 16)] = e0 + 1
        else:
            mask = jnp.uint32(lv["size"] - 1)
            off = jnp.int32(2 * lv["row_off"])
            lo = [pg[d].astype(jnp.uint32) * jnp.uint32(_PRIMES[d])
                  for d in range(4)]
            hi = [lo[d] + jnp.uint32(_PRIMES[d]) for d in range(4)]
            for c in range(16):
                t = [hi[d] if (c >> d) & 1 else lo[d] for d in range(4)]
                h = ((t[0] ^ t[1]) ^ (t[2] ^ t[3])) & mask
                e0 = h.astype(jnp.int32) * 2 + off
                idxs[fb][pl.ds(c * _P + b16, 16)] = e0
                idxs[fb][pl.ds(_CP + c * _P + b16, 16)] = e0 + 1
        return c2

    def pass_b(g, c2, s, fb):
        b16 = g * 16
        fr = [frac_v[fb, d, pl.ds(b16, 16)] for d in range(4)]
        om = [1.0 - f for f in fr]
        u = [om[0] * om[1], fr[0] * om[1], om[0] * fr[1], fr[0] * fr[1]]
        v = [om[2] * om[3], fr[2] * om[3], om[2] * fr[3], fr[2] * fr[3]]
        acc0 = jnp.zeros((16,), jnp.float32)
        acc1 = jnp.zeros((16,), jnp.float32)
        rv = rows[fb]
        for c in range(16):
            w = u[c & 3] * v[c >> 2]
            g0 = rv[pl.ds(c * _P + b16, 16)]
            g1 = rv[pl.ds(_CP + c * _P + b16, 16)]
            acc0 = acc0 + w * g0
            acc1 = acc1 + w * g1
        oidx = (b16 + iota) * 32 + (2 * s)
        plsc.store_scatter(out_v, [oidx], acc0)
        plsc.store_scatter(out_v, [oidx + 1], acc1)
        return c2

    def chunk_body(ci, carry):
        pt0 = (base_chunk + ci) * _P
        pltpu.sync_copy(xyzt_hbm.at[:, pl.ds(pt0, _P)], xyzt_v)
        lax.fori_loop(0, _G, functools.partial(build_a, lv=_LV[0], fb=0), 0,
                      unroll=False)
        cp = pltpu.async_copy(table_hbm.at[idx0_v], rows0_v, sem0)
        for s in range(1, _NUM_SCALES + 1):
            fb, pb = s % 2, (s - 1) % 2
            if s < _NUM_SCALES:
                lax.fori_loop(0, _G,
                              functools.partial(build_a, lv=_LV[s], fb=fb), 0,
                              unroll=False)
                nxt = pltpu.async_copy(table_hbm.at[idxs[fb]],
                                       rows[fb], sems[fb])
            cp.wait()
            lax.fori_loop(0, _G, functools.partial(pass_b, s=s - 1, fb=pb), 0,
                          unroll=False)
            if s < _NUM_SCALES:
                cp = nxt
        pltpu.sync_copy(out_v, out_hbm.at[pl.ds(pt0 * 32, _P * 32)])
        return carry

    lax.fori_loop(0, my_chunks, chunk_body, 0, unroll=False)


@jax.jit
def _encode(xyzt_t, table2):
    mesh = plsc.VectorSubcoreMesh(core_axis_name="c", subcore_axis_name="s",
                                  num_cores=_NC, num_subcores=_NS)
    f = pl.kernel(
        _tec_kernel,
        out_type=jax.ShapeDtypeStruct((_NPAD * 2 * _NUM_SCALES,), jnp.float32),
        mesh=mesh,
        compiler_params=pltpu.CompilerParams(needs_layout_passes=False),
        scratch_types=[
            pltpu.VMEM((4, _P), jnp.float32),
            pltpu.VMEM((2, 4, _P), jnp.float32),
            pltpu.VMEM((2 * _CP,), jnp.int32),
            pltpu.VMEM((2 * _CP,), jnp.int32),
            pltpu.VMEM((2 * _CP,), jnp.float32),
            pltpu.VMEM((2 * _CP,), jnp.float32),
            pltpu.VMEM((_P * 2 * _NUM_SCALES,), jnp.float32),
            pltpu.SemaphoreType.DMA,
            pltpu.SemaphoreType.DMA,
        ],
    )
    return f(xyzt_t, table2)


def kernel(xyzt, hash_table):
    n = xyzt.shape[0]
    xyzt_t = jnp.zeros((4, _NPAD), jnp.float32).at[:, :n].set(xyzt.T)
    out = _encode(xyzt_t, hash_table)
    return out.reshape(_NPAD, 2 * _NUM_SCALES)[:n]


# speedup vs baseline: 13.8082x; 1.2377x over previous
import functools

import numpy as np
import jax
import jax.numpy as jnp
from jax import lax
from jax.experimental import pallas as pl
from jax.experimental.pallas import tpu as pltpu, tpu_sc as plsc

_MIN_RES = np.array([16, 16, 16, 16], dtype=np.float64)
_MAX_RES = np.array([256, 256, 256, 128], dtype=np.float64)
_NUM_SCALES = 16
_MAX_PARAMS = 2 ** 19
_F = 2
_PRIMES = (1, 2654435761, 805459861, 3674653429)


def _levels():
    b = np.exp((np.log(_MAX_RES) - np.log(_MIN_RES)) / (_NUM_SCALES - 1))
    out, total = [], 0
    for i in range(_NUM_SCALES):
        res = np.ceil(_MIN_RES * np.power(b, i)).astype(np.int64)
        raw = int(res[0] + 1) * int(res[1] + 1) * int(res[2] + 1) * int(res[3] + 1)
        p = raw if raw % 8 == 0 else (raw + 7) // 8 * 8
        p = min(_MAX_PARAMS, p)
        out.append(dict(res=tuple(int(r) for r in res), size=p, tiled=raw <= p,
                        off=total))
        total += p * _F
    return out, total


_LV, _TOTAL = _levels()

_NC = 2
_NS = 16
_NW = _NC * _NS
_P = 256
_G = _P // 16
_CP = 16 * _P
_N = 1000000
_NCHUNK = -(-_N // (_NW * _P))
_NPAD = _NW * _NCHUNK * _P
_NCHUNK0 = 160
_NCHUNK1 = 2 * _NCHUNK - _NCHUNK0
_HMASK = _MAX_PARAMS - 1
_SPW = 2 * _MAX_PARAMS
_HOFF = _LV[3]["off"]
_BW = 16384


def _tec_kernel(xyzt_hbm, table_hbm, prm_hbm, out_hbm,
                xyzt0_v, xyzt1_v, frac0_v, frac1_v, idx0_v, idx1_v,
                rows0_v, rows1_v, feat_v, prm_v, bounce_v, sem0, sem1, spt):
    cid = lax.axis_index("c")
    sid = lax.axis_index("s")
    base_chunk = jnp.where(cid == 0, sid * _NCHUNK0,
                           16 * _NCHUNK0 + sid * _NCHUNK1)
    my_chunks = jnp.where(cid == 0, _NCHUNK0, _NCHUNK1)
    iota = lax.iota(jnp.int32, 16)
    xyzts = (xyzt0_v, xyzt1_v)
    idxs = (idx0_v, idx1_v)
    rows = (rows0_v, rows1_v)
    fracs = (frac0_v, frac1_v)
    sems = (sem0, sem1)

    pltpu.sync_copy(prm_hbm, prm_v)

    def build(g, c2, resf, half, stat_lv):
        b16 = g * 16
        pg = []
        for d in range(4):
            pos = xyzts[half][d, pl.ds(b16, 16)] * resf[d]
            pgi = pos.astype(jnp.int32)
            fracs[half][d, pl.ds(b16, 16)] = pos - pgi.astype(jnp.float32)
            pg.append(pgi)
        if stat_lv is not None:
            r = stat_lv["res"]
            st = (1, r[0] + 1, (r[0] + 1) * (r[1] + 1),
                  (r[0] + 1) * (r[1] + 1) * (r[2] + 1))
            lo = [pg[d] * jnp.int32(2 * st[d]) for d in range(4)]
            hi = [lo[d] + jnp.int32(2 * st[d]) for d in range(4)]
            for c in range(16):
                t = [hi[d] if (c >> d) & 1 else lo[d] for d in range(4)]
                e0 = (t[0] + t[1]) + (t[2] + t[3])
                idxs[half][pl.ds(c * _P + b16, 16)] = e0
                idxs[half][pl.ds(_CP + c * _P + b16, 16)] = e0 + 1
        else:
            mask = jnp.uint32(_HMASK)
            lo = [pg[d].astype(jnp.uint32) * jnp.uint32(_PRIMES[d])
                  for d in range(4)]
            hi = [lo[d] + jnp.uint32(_PRIMES[d]) for d in range(4)]
            for c in range(16):
                t = [hi[d] if (c >> d) & 1 else lo[d] for d in range(4)]
                h = ((t[0] ^ t[1]) ^ (t[2] ^ t[3])) & mask
                e0 = h.astype(jnp.int32) * 2
                idxs[half][pl.ds(c * _P + b16, 16)] = e0
                idxs[half][pl.ds(_CP + c * _P + b16, 16)] = e0 + 1
        return c2

    def pass_b(g, c2, half):
        b16 = g * 16
        fr = [fracs[half][d, pl.ds(b16, 16)] for d in range(4)]
        om = [1.0 - f for f in fr]
        u = [om[0] * om[1], fr[0] * om[1], om[0] * fr[1], fr[0] * fr[1]]
        v = [om[2] * om[3], fr[2] * om[3], om[2] * fr[3], fr[2] * fr[3]]
        acc0 = jnp.zeros((16,), jnp.float32)
        acc1 = jnp.zeros((16,), jnp.float32)
        rv = rows[half]
        for c in range(16):
            w = u[c & 3] * v[c >> 2]
            g0 = rv[pl.ds(c * _P + b16, 16)]
            g1 = rv[pl.ds(_CP + c * _P + b16, 16)]
            acc0 = acc0 + w * g0
            acc1 = acc1 + w * g1
        oidx = (half * _P + b16 + iota) * 2
        plsc.store_scatter(feat_v, [oidx], acc0)
        plsc.store_scatter(feat_v, [oidx + 1], acc1)
        return c2

    def run_level(s_dyn, resf, off_words, stat_lv, stage_words):
        n_stage = -(-stage_words // (16 * _BW))
        sl_pad = n_stage * _BW
        for i in range(n_stage):
            boff = sid * sl_pad + i * _BW
            pltpu.sync_copy(table_hbm.at[pl.ds(off_words + boff, _BW)],
                            bounce_v)
            pltpu.sync_copy(bounce_v, spt.at[pl.ds(boff, _BW)])
        plsc.subcore_barrier()

        def body(k, c2):
            pt0 = (base_chunk + 2 * k) * _P
            pltpu.sync_copy(xyzt_hbm.at[:, pl.ds(pt0, _P)], xyzt0_v)
            lax.fori_loop(0, _G, functools.partial(
                build, resf=resf, half=0, stat_lv=stat_lv), 0, unroll=False)
            cp0 = pltpu.async_copy(spt.at[idx0_v], rows0_v, sem0)
            pltpu.sync_copy(xyzt_hbm.at[:, pl.ds(pt0 + _P, _P)], xyzt1_v)
            lax.fori_loop(0, _G, functools.partial(
                build, resf=resf, half=1, stat_lv=stat_lv), 0, unroll=False)
            cp1 = pltpu.async_copy(spt.at[idx1_v], rows1_v, sem1)
            cp0.wait()
            lax.fori_loop(0, _G, functools.partial(pass_b, half=0), 0,
                          unroll=False)
            cp1.wait()
            lax.fori_loop(0, _G, functools.partial(pass_b, half=1), 0,
                          unroll=False)
            pltpu.sync_copy(
                feat_v,
                out_hbm.at[pl.ds(s_dyn * (2 * _NPAD) + pt0 * 2, 4 * _P)])
            return c2

        lax.fori_loop(0, my_chunks // 2, body, 0, unroll=False)
        plsc.subcore_barrier()

    for s in range(3):
        lv = _LV[s]
        run_level(jnp.int32(s), [np.float32(r) for r in lv["res"]],
                  jnp.int32(lv["off"]), lv, lv["size"] * 2)

    def hashed_level(j, c2):
        prow = prm_v[j, pl.ds(0, 16)].astype(jnp.float32)
        dn = lax.GatherDimensionNumbers(offset_dims=(),
                                        collapsed_slice_dims=(0,),
                                        start_index_map=(0,))
        resf = [lax.gather(prow, jnp.full((16, 1), d, jnp.int32), dn, (1,),
                           mode=lax.GatherScatterMode.PROMISE_IN_BOUNDS)
                for d in range(4)]
        run_level(j + 3, resf, _HOFF + j * _SPW, None, _SPW)
        return c2

    lax.fori_loop(0, 13, hashed_level, 0, unroll=False)


@jax.jit
def _encode(xyzt_t, table, prm):
    mesh = plsc.VectorSubcoreMesh(core_axis_name="c", subcore_axis_name="s",
                                  num_cores=_NC, num_subcores=_NS)
    f = pl.kernel(
        _tec_kernel,
        out_type=jax.ShapeDtypeStruct((_NUM_SCALES * _NPAD * 2,), jnp.float32),
        mesh=mesh,
        compiler_params=pltpu.CompilerParams(needs_layout_passes=False),
        scratch_types=[
            pltpu.VMEM((4, _P), jnp.float32),
            pltpu.VMEM((4, _P), jnp.float32),
            pltpu.VMEM((4, _P), jnp.float32),
            pltpu.VMEM((4, _P), jnp.float32),
            pltpu.VMEM((2 * _CP,), jnp.int32),
            pltpu.VMEM((2 * _CP,), jnp.int32),
            pltpu.VMEM((2 * _CP,), jnp.float32),
            pltpu.VMEM((2 * _CP,), jnp.float32),
            pltpu.VMEM((4 * _P,), jnp.float32),
            pltpu.VMEM((13, 16), jnp.int32),
            pltpu.VMEM((_BW,), jnp.float32),
            pltpu.SemaphoreType.DMA,
            pltpu.SemaphoreType.DMA,
            pltpu.VMEM_SHARED((_SPW,), jnp.float32),
        ],
    )
    return f(xyzt_t, table, prm)


_PRM = np.zeros((13, 16), np.int32)
for _j, _lv in enumerate(_LV[3:]):
    _PRM[_j, :4] = _lv["res"]


def kernel(xyzt, hash_table):
    n = xyzt.shape[0]
    xyzt_t = jnp.zeros((4, _NPAD), jnp.float32).at[:, :n].set(xyzt.T)
    table_pad = jnp.concatenate([hash_table,
                                 jnp.zeros((16 * _BW,), jnp.float32)])
    out = _encode(xyzt_t, table_pad, jnp.asarray(_PRM))
    out = out.reshape(_NUM_SCALES, _NPAD, 2)[:, :n, :]
    return out.transpose(1, 0, 2).reshape(n, 2 * _NUM_SCALES)


# speedup vs baseline: 38.7405x; 2.8056x over previous
import functools

import numpy as np
import jax
import jax.numpy as jnp
from jax import lax
from jax.experimental import pallas as pl
from jax.experimental.pallas import tpu as pltpu, tpu_sc as plsc

_MIN_RES = np.array([16, 16, 16, 16], dtype=np.float64)
_MAX_RES = np.array([256, 256, 256, 128], dtype=np.float64)
_NUM_SCALES = 16
_MAX_PARAMS = 2 ** 19
_F = 2
_PRIMES = (1, 2654435761, 805459861, 3674653429)


def _levels():
    b = np.exp((np.log(_MAX_RES) - np.log(_MIN_RES)) / (_NUM_SCALES - 1))
    out, total = [], 0
    for i in range(_NUM_SCALES):
        res = np.ceil(_MIN_RES * np.power(b, i)).astype(np.int64)
        raw = int(res[0] + 1) * int(res[1] + 1) * int(res[2] + 1) * int(res[3] + 1)
        p = raw if raw % 8 == 0 else (raw + 7) // 8 * 8
        p = min(_MAX_PARAMS, p)
        out.append(dict(res=tuple(int(r) for r in res), size=p, tiled=raw <= p,
                        off=total))
        total += p * _F
    return out, total


_LV, _TOTAL = _levels()

_NC = 2
_NS = 16
_NW = _NC * _NS
_P = 256
_G = _P // 16
_CP = 16 * _P
_N = 1000000
_NCHUNK = -(-_N // (_NW * _P))
_NPAD = _NW * _NCHUNK * _P
_NCHUNK0 = 123
_NCHUNK1 = 2 * _NCHUNK - _NCHUNK0
_HMASK = _MAX_PARAMS - 1
_SPW = 2 * _MAX_PARAMS
_HOFF = _LV[3]["off"]
_BW = 8192


def _tec_kernel(xyzt_hbm, table_hbm, prm_hbm, out_hbm,
                xyzt0_v, xyzt1_v, xyzt2_v, frac0_v, frac1_v, frac2_v,
                idx0_v, idx1_v, idx2_v, rows0_v, rows1_v, rows2_v,
                feat_v, prm_v, sem0, sem1, sem2, spt):
    cid = lax.axis_index("c")
    sid = lax.axis_index("s")
    base_chunk = jnp.where(cid == 0, sid * _NCHUNK0,
                           16 * _NCHUNK0 + sid * _NCHUNK1)
    my_chunks = jnp.where(cid == 0, _NCHUNK0, _NCHUNK1)
    iota = lax.iota(jnp.int32, 16)
    xyzts = (xyzt0_v, xyzt1_v, xyzt2_v)
    idxs = (idx0_v, idx1_v, idx2_v)
    rows = (rows0_v, rows1_v, rows2_v)
    fracs = (frac0_v, frac1_v, frac2_v)
    sems = (sem0, sem1, sem2)

    pltpu.sync_copy(prm_hbm, prm_v)

    def build(g, c2, resf, half, stat_lv):
        b16 = g * 16
        pg = []
        for d in range(4):
            pos = xyzts[half][d, pl.ds(b16, 16)] * resf[d]
            pgi = pos.astype(jnp.int32)
            fracs[half][d, pl.ds(b16, 16)] = pos - pgi.astype(jnp.float32)
            pg.append(pgi)
        if stat_lv is not None:
            r = stat_lv["res"]
            st = (1, r[0] + 1, (r[0] + 1) * (r[1] + 1),
                  (r[0] + 1) * (r[1] + 1) * (r[2] + 1))
            lo = [pg[d] * jnp.int32(2 * st[d]) for d in range(4)]
            hi = [lo[d] + jnp.int32(2 * st[d]) for d in range(4)]
            for c in range(16):
                t = [hi[d] if (c >> d) & 1 else lo[d] for d in range(4)]
                e0 = (t[0] + t[1]) + (t[2] + t[3])
                idxs[half][pl.ds(c * _P + b16, 16)] = e0
                idxs[half][pl.ds(_CP + c * _P + b16, 16)] = e0 + 1
        else:
            mask = jnp.uint32(_HMASK)
            lo = [pg[d].astype(jnp.uint32) * jnp.uint32(_PRIMES[d])
                  for d in range(4)]
            hi = [lo[d] + jnp.uint32(_PRIMES[d]) for d in range(4)]
            for c in range(16):
                t = [hi[d] if (c >> d) & 1 else lo[d] for d in range(4)]
                h = ((t[0] ^ t[1]) ^ (t[2] ^ t[3])) & mask
                e0 = h.astype(jnp.int32) * 2
                idxs[half][pl.ds(c * _P + b16, 16)] = e0
                idxs[half][pl.ds(_CP + c * _P + b16, 16)] = e0 + 1
        return c2

    def pass_b(g, c2, half):
        b16 = g * 16
        fr = [fracs[half][d, pl.ds(b16, 16)] for d in range(4)]
        om = [1.0 - f for f in fr]
        u = [om[0] * om[1], fr[0] * om[1], om[0] * fr[1], fr[0] * fr[1]]
        v = [om[2] * om[3], fr[2] * om[3], om[2] * fr[3], fr[2] * fr[3]]
        acc0 = jnp.zeros((16,), jnp.float32)
        acc1 = jnp.zeros((16,), jnp.float32)
        rv = rows[half]
        for c in range(16):
            w = u[c & 3] * v[c >> 2]
            g0 = rv[pl.ds(c * _P + b16, 16)]
            g1 = rv[pl.ds(_CP + c * _P + b16, 16)]
            acc0 = acc0 + w * g0
            acc1 = acc1 + w * g1
        feat_v[pl.ds(half * _P + b16, 16)] = acc0
        feat_v[pl.ds((3 + half) * _P + b16, 16)] = acc1
        return c2

    def run_level(s_dyn, resf, off_words, stat_lv, stage_words):
        n_stage = -(-stage_words // (16 * _BW))
        sl_pad = n_stage * _BW
        for i in range(n_stage):
            boff = sid * sl_pad + i * _BW
            pltpu.sync_copy(table_hbm.at[pl.ds(off_words + boff, _BW)],
                            rows0_v)
            pltpu.sync_copy(rows0_v, spt.at[pl.ds(boff, _BW)])
        plsc.subcore_barrier()

        def stage_j(pt0, j):
            pltpu.sync_copy(xyzt_hbm.at[:, pl.ds(pt0 + j * _P, _P)], xyzts[j])
            lax.fori_loop(0, _G, functools.partial(
                build, resf=resf, half=j, stat_lv=stat_lv), 0, unroll=False)
            return pltpu.async_copy(spt.at[idxs[j]], rows[j], sems[j])

        def drain_j(pt0, j, cp):
            cp.wait()
            lax.fori_loop(0, _G, functools.partial(pass_b, half=j), 0,
                          unroll=False)
            pltpu.sync_copy(
                feat_v.at[pl.ds(j * _P, _P)],
                out_hbm.at[pl.ds(s_dyn * (2 * _NPAD) + pt0 + j * _P, _P)])
            pltpu.sync_copy(
                feat_v.at[pl.ds((3 + j) * _P, _P)],
                out_hbm.at[pl.ds(s_dyn * (2 * _NPAD) + _NPAD + pt0 + j * _P,
                                 _P)])

        def body(k, c2):
            pt0 = (base_chunk + 3 * k) * _P
            cp0 = stage_j(pt0, 0)
            cp1 = stage_j(pt0, 1)
            drain_j(pt0, 0, cp0)
            cp2 = stage_j(pt0, 2)
            drain_j(pt0, 1, cp1)
            drain_j(pt0, 2, cp2)
            return c2

        lax.fori_loop(0, my_chunks // 3, body, 0, unroll=False)
        plsc.subcore_barrier()

    for s in range(3):
        lv = _LV[s]
        run_level(jnp.int32(s), [np.float32(r) for r in lv["res"]],
                  jnp.int32(lv["off"]), lv, lv["size"] * 2)

    def hashed_level(j, c2):
        prow = prm_v[j, pl.ds(0, 16)].astype(jnp.float32)
        dn = lax.GatherDimensionNumbers(offset_dims=(),
                                        collapsed_slice_dims=(0,),
                                        start_index_map=(0,))
        resf = [lax.gather(prow, jnp.full((16, 1), d, jnp.int32), dn, (1,),
                           mode=lax.GatherScatterMode.PROMISE_IN_BOUNDS)
                for d in range(4)]
        run_level(j + 3, resf, _HOFF + j * _SPW, None, _SPW)
        return c2

    lax.fori_loop(0, 13, hashed_level, 0, unroll=False)


@jax.jit
def _encode(xyzt_t, table, prm):
    mesh = plsc.VectorSubcoreMesh(core_axis_name="c", subcore_axis_name="s",
                                  num_cores=_NC, num_subcores=_NS)
    f = pl.kernel(
        _tec_kernel,
        out_type=jax.ShapeDtypeStruct((_NUM_SCALES * 2 * _NPAD,), jnp.float32),
        mesh=mesh,
        compiler_params=pltpu.CompilerParams(needs_layout_passes=False),
        scratch_types=[
            pltpu.VMEM((4, _P), jnp.float32),
            pltpu.VMEM((4, _P), jnp.float32),
            pltpu.VMEM((4, _P), jnp.float32),
            pltpu.VMEM((4, _P), jnp.float32),
            pltpu.VMEM((4, _P), jnp.float32),
            pltpu.VMEM((4, _P), jnp.float32),
            pltpu.VMEM((2 * _CP,), jnp.int32),
            pltpu.VMEM((2 * _CP,), jnp.int32),
            pltpu.VMEM((2 * _CP,), jnp.int32),
            pltpu.VMEM((2 * _CP,), jnp.float32),
            pltpu.VMEM((2 * _CP,), jnp.float32),
            pltpu.VMEM((2 * _CP,), jnp.float32),
            pltpu.VMEM((6 * _P,), jnp.float32),
            pltpu.VMEM((13, 16), jnp.int32),
            pltpu.SemaphoreType.DMA,
            pltpu.SemaphoreType.DMA,
            pltpu.SemaphoreType.DMA,
            pltpu.VMEM_SHARED((_SPW,), jnp.float32),
        ],
    )
    return f(xyzt_t, table, prm)


_PRM = np.zeros((13, 16), np.int32)
for _j, _lv in enumerate(_LV[3:]):
    _PRM[_j, :4] = _lv["res"]


_P2 = 1968
_C2 = _NPAD // _NW // _P2


def _ilv_kernel(lvl_hbm, out_hbm, stage_v, out_v, sem0):
    cid = lax.axis_index("c")
    sid = lax.axis_index("s")
    wid = sid * _NC + cid
    iota = lax.iota(jnp.int32, 16)
    oidx_base = iota * 32

    def chunk_body(ci, carry):
        pt0 = (wid * _C2 + ci) * _P2
        cps = []
        for b in range(32):
            cps.append(pltpu.async_copy(
                lvl_hbm.at[pl.ds(b * _NPAD + pt0, _P2)],
                stage_v.at[pl.ds(b * _P2, _P2)], sem0))
        for cp in cps:
            cp.wait()

        def grp(g, c2):
            b16 = g * 16
            for b in range(32):
                s, f = b >> 1, b & 1
                x = stage_v[pl.ds(b * _P2 + b16, 16)]
                plsc.store_scatter(out_v, [oidx_base + (b16 * 32 + 2 * s + f)], x)
            return c2

        lax.fori_loop(0, _P2 // 16, grp, 0, unroll=False)
        pltpu.sync_copy(out_v, out_hbm.at[pl.ds(pt0 * 32, _P2 * 32)])
        return carry

    lax.fori_loop(0, _C2, chunk_body, 0, unroll=False)


@jax.jit
def _interleave(lvl):
    mesh = plsc.VectorSubcoreMesh(core_axis_name="c", subcore_axis_name="s",
                                  num_cores=_NC, num_subcores=_NS)
    f = pl.kernel(
        _ilv_kernel,
        out_type=jax.ShapeDtypeStruct((_NPAD * 32,), jnp.float32),
        mesh=mesh,
        compiler_params=pltpu.CompilerParams(needs_layout_passes=False),
        scratch_types=[
            pltpu.VMEM((32 * _P2,), jnp.float32),
            pltpu.VMEM((32 * _P2,), jnp.float32),
            pltpu.SemaphoreType.DMA,
        ],
    )
    return f(lvl)


def kernel(xyzt, hash_table):
    n = xyzt.shape[0]
    xyzt_t = jnp.zeros((4, _NPAD), jnp.float32).at[:, :n].set(xyzt.T)
    lvl = _encode(xyzt_t, hash_table, jnp.asarray(_PRM))
    out = _interleave(lvl)
    return out.reshape(_NPAD, 2 * _NUM_SCALES)[:n]
